# baseline (speedup 1.0000x reference)
import jax
import jax.numpy as jnp
from jax import lax
from jax.experimental import pallas as pl
from jax.experimental.pallas import tpu as pltpu
from jax.experimental.pallas import tpu_sc as plsc

_N = 4096
_NC = 2
_NS = 16
_NW = _NC * _NS
_RPW = _N // _NW
_CHUNK = 16
_NCHUNK = _RPW // _CHUNK
_L = 16
_NBLK = _N // _L
_UNROLL = 8


def _dyn_gather(x, idx):
    dnums = lax.GatherDimensionNumbers(
        offset_dims=(), collapsed_slice_dims=(0,), start_index_map=(0,)
    )
    return lax.gather(
        x,
        idx.reshape(_L, 1),
        dnums,
        slice_sizes=(1,),
        mode=lax.GatherScatterMode.PROMISE_IN_BOUNDS,
    )


def _sc_body(adj_hbm, theta_hbm, out_hbm, buf, theta_v):
    c = lax.axis_index("c")
    s = lax.axis_index("s")
    base = (s * _NC + c) * _RPW
    pltpu.sync_copy(theta_hbm.at[pl.ds(base, _RPW)], theta_v)
    iota = lax.iota(jnp.int32, _L)
    zero = jnp.zeros((_L,), jnp.float32)

    def chunk_body(ch, carry):
        row0 = base + ch * _CHUNK
        pltpu.sync_copy(adj_hbm.at[pl.ds(row0, _CHUNK)], buf)

        tvec = theta_v[pl.ds(ch * _CHUNK, _CHUNK)]

        def row_body(r, rcarry):
            i = row0 + r
            def sum_body(ob, acc):
                col0 = ob * (_L * _UNROLL)
                for u in range(_UNROLL):
                    acc = acc + buf[r, pl.ds(col0 + u * _L, _L)]
                return acc
            acc = lax.fori_loop(0, _NBLK // _UNROLL, sum_body, zero)
            db16 = (i // _L) * _L
            lane_mask = iota == (i % _L)
            vdb = buf[r, pl.ds(db16, _L)]
            acc = acc - jnp.where(lane_mask, vdb, zero)
            for sh in (1, 2, 4, 8):
                acc = acc + _dyn_gather(acc, iota ^ sh)
            cnt = acc + jnp.float32(1e-10)
            th = _dyn_gather(tvec, jnp.full((_L,), r, jnp.int32))
            nbw = th / cnt

            def w_body(ob, wcarry):
                col0 = ob * (_L * _UNROLL)
                for u in range(_UNROLL):
                    sl = pl.ds(col0 + u * _L, _L)
                    v = buf[r, sl]
                    buf[r, sl] = jnp.where(v != jnp.float32(0.0), nbw, zero)
                return wcarry

            lax.fori_loop(0, _NBLK // _UNROLL, w_body, 0)
            vdb2 = buf[r, pl.ds(db16, _L)]
            buf[r, pl.ds(db16, _L)] = jnp.where(lane_mask, th, vdb2)
            return rcarry

        lax.fori_loop(0, _CHUNK, row_body, 0)
        pltpu.sync_copy(buf, out_hbm.at[pl.ds(row0, _CHUNK)])
        return carry

    lax.fori_loop(0, _NCHUNK, chunk_body, 0)


def kernel(adj, theta):
    mesh = plsc.VectorSubcoreMesh(core_axis_name="c", subcore_axis_name="s")
    f = pl.kernel(
        _sc_body,
        out_type=jax.ShapeDtypeStruct((_N, _N), jnp.float32),
        mesh=mesh,
        scratch_types=[
            pltpu.VMEM((_CHUNK, _N), jnp.float32),
            pltpu.VMEM((_RPW,), jnp.float32),
        ],
    )
    return f(adj, theta.reshape(_N))
